# Initial kernel scaffold; baseline (speedup 1.0000x reference)
import jax
import jax.numpy as jnp
from jax.experimental import pallas as pl


def kernel(inputs, token_table, pos_table):
    raise NotImplementedError("write your pallas kernel here")


# speedup vs baseline: 3.1072x; 3.1072x over previous
import functools

import jax
import jax.numpy as jnp
from jax import lax
from jax.experimental import pallas as pl
from jax.experimental.pallas import tpu as pltpu
from jax.experimental.pallas import tpu_sc as plsc

SEQ = 200
NSEQ = 4096
DIM = 64
NC = 2
NS = 16
NW = NC * NS
SEQ_PER_W = NSEQ // NW


def _body(idx_hbm, tok_hbm, pos_hbm, out_hbm, pos_v, idx_v, rows_v, sem):
    wid = lax.axis_index("s") * NC + lax.axis_index("c")
    pltpu.sync_copy(pos_hbm, pos_v)

    def seq_body(s, carry):
        base = (wid * SEQ_PER_W + s) * SEQ
        pltpu.sync_copy(idx_hbm.at[pl.ds(base, SEQ)], idx_v)
        cp1 = pltpu.async_copy(
            tok_hbm.at[idx_v.at[pl.ds(0, 128)]], rows_v.at[pl.ds(0, 128)], sem)
        cp2 = pltpu.async_copy(
            tok_hbm.at[idx_v.at[pl.ds(128, 72)]], rows_v.at[pl.ds(128, 72)], sem)
        cp1.wait()
        cp2.wait()

        def add_row(i, c):
            for j in range(DIM // 16):
                sl = pl.ds(j * 16, 16)
                plsc.addupdate(rows_v.at[i, sl], pos_v[i, sl])
            return c

        lax.fori_loop(0, SEQ, add_row, 0)
        pltpu.sync_copy(rows_v, out_hbm.at[pl.ds(base, SEQ)])
        return carry

    lax.fori_loop(0, SEQ_PER_W, seq_body, 0)


def kernel(inputs, token_table, pos_table):
    idx = inputs.reshape(-1).astype(jnp.int32)
    mesh = plsc.VectorSubcoreMesh(core_axis_name="c", subcore_axis_name="s")
    run = functools.partial(
        pl.kernel,
        mesh=mesh,
        compiler_params=pltpu.CompilerParams(use_tc_tiling_on_sc=False),
        out_type=jax.ShapeDtypeStruct((NSEQ * SEQ, DIM), jnp.float32),
        scratch_types=[
            pltpu.VMEM((SEQ, DIM), jnp.float32),
            pltpu.VMEM((SEQ,), jnp.int32),
            pltpu.VMEM((SEQ, DIM), jnp.float32),
            pltpu.SemaphoreType.DMA,
        ],
    )(_body)
    out = run(idx, token_table, pos_table)
    return out.reshape(NSEQ, SEQ, DIM)


# speedup vs baseline: 4.1294x; 1.3290x over previous
import functools

import jax
import jax.numpy as jnp
from jax import lax
from jax.experimental import pallas as pl
from jax.experimental.pallas import tpu as pltpu
from jax.experimental.pallas import tpu_sc as plsc

SEQ = 200
NSEQ = 4096
DIM = 64
NC = 2
NS = 16
NW = NC * NS
SEQ_PER_W = NSEQ // NW
NBUF = 4


def _body(idx_hbm, tok_hbm, pos_hbm, out_hbm,
          pos_v, idx_v, rows0, rows1, rows2, rows3,
          gsem, osem):
    rows = (rows0, rows1, rows2, rows3)
    wid = lax.axis_index("s") * NC + lax.axis_index("c")
    wbase = wid * SEQ_PER_W * SEQ
    pltpu.sync_copy(pos_hbm, pos_v)
    pltpu.sync_copy(idx_hbm.at[pl.ds(wbase, SEQ_PER_W * SEQ)], idx_v)

    def gather_cps(s, b):
        off = s * SEQ
        return (
            pltpu.make_async_copy(
                tok_hbm.at[idx_v.at[pl.ds(off, 128)]],
                rows[b].at[pl.ds(0, 128)], gsem[b]),
            pltpu.make_async_copy(
                tok_hbm.at[idx_v.at[pl.ds(off + 128, 72)]],
                rows[b].at[pl.ds(128, 72)], gsem[b]),
        )

    def fire_gather(s, b):
        for cp in gather_cps(s, b):
            cp.start()

    def wait_gather(s, b):
        for cp in gather_cps(s, b):
            cp.wait()

    def out_cp(s, b):
        return pltpu.make_async_copy(
            rows[b], out_hbm.at[pl.ds(wbase + s * SEQ, SEQ)], osem[b])

    def add_pos(b):
        def add_row(i, c):
            for j in range(DIM // 16):
                sl = pl.ds(j * 16, 16)
                plsc.addupdate(rows[b].at[i, sl], pos_v[i, sl])
            return c
        lax.fori_loop(0, SEQ, add_row, 0)

    def step(s, b, drain_out, next_gather):
        wait_gather(s, b)
        add_pos(b)
        out_cp(s, b).start()
        bn = (b + 2) % NBUF
        if drain_out:
            out_cp(s - 2, bn).wait()
        if next_gather:
            fire_gather(s + 2, bn)

    fire_gather(0, 0)
    fire_gather(1, 1)
    step(0, 0, False, True)
    step(1, 1, False, True)
    step(2, 2, True, True)
    step(3, 3, True, True)

    def group(g, carry):
        s = g * NBUF
        for b in range(NBUF):
            step(s + b, b, True, True)
        return carry

    lax.fori_loop(1, SEQ_PER_W // NBUF - 1, group, 0)

    t = SEQ_PER_W - NBUF
    step(t + 0, 0, True, True)
    step(t + 1, 1, True, True)
    step(t + 2, 2, False, False)
    step(t + 3, 3, False, False)
    for b in range(NBUF):
        out_cp(t + b, b).wait()


def kernel(inputs, token_table, pos_table):
    idx = inputs.reshape(-1).astype(jnp.int32)
    mesh = plsc.VectorSubcoreMesh(core_axis_name="c", subcore_axis_name="s")
    run = functools.partial(
        pl.kernel,
        mesh=mesh,
        compiler_params=pltpu.CompilerParams(use_tc_tiling_on_sc=False),
        out_type=jax.ShapeDtypeStruct((NSEQ * SEQ, DIM), jnp.float32),
        scratch_types=[
            pltpu.VMEM((SEQ, DIM), jnp.float32),
            pltpu.VMEM((SEQ_PER_W * SEQ,), jnp.int32),
            pltpu.VMEM((SEQ, DIM), jnp.float32),
            pltpu.VMEM((SEQ, DIM), jnp.float32),
            pltpu.VMEM((SEQ, DIM), jnp.float32),
            pltpu.VMEM((SEQ, DIM), jnp.float32),
            [pltpu.SemaphoreType.DMA] * NBUF,
            [pltpu.SemaphoreType.DMA] * NBUF,
        ],
    )(_body)
    out = run(idx, token_table, pos_table)
    return out.reshape(NSEQ, SEQ, DIM)
